# baseline (speedup 1.0000x reference)
import functools

import jax
import jax.numpy as jnp
from jax import lax
from jax.experimental import pallas as pl
from jax.experimental.pallas import tpu as pltpu
from jax.experimental.pallas import tpu_sc as plsc

NC = 2
NS = 16
NW = NC * NS

B = 4
S = 8192
D = 1024
ROWS_PER_W = S // NW
CHUNK = 32

STREAM_ROWS = 192
SPMEM_ROWS = ROWS_PER_W - STREAM_ROWS
N_ST = STREAM_ROWS // CHUNK
N_SP = SPMEM_ROWS // CHUNK


def _make_sc_copy():
    mesh = plsc.VectorSubcoreMesh(core_axis_name="c", subcore_axis_name="s")

    @functools.partial(
        pl.kernel,
        out_type=jax.ShapeDtypeStruct((B, S, D), jnp.float32),
        mesh=mesh,
        scratch_types=[
            pltpu.VMEM((CHUNK, D), jnp.float32),
            pltpu.VMEM((CHUNK, D), jnp.float32),
            pltpu.VMEM_SHARED((NS, SPMEM_ROWS, D), jnp.float32),
            pltpu.SemaphoreType.DMA,
            pltpu.SemaphoreType.DMA,
            pltpu.SemaphoreType.DMA,
            pltpu.SemaphoreType.DMA,
            pltpu.SemaphoreType.DMA,
            pltpu.SemaphoreType.DMA,
        ],
    )
    def body(table_hbm, out_hbm, buf0, buf1, shb,
             isem0, isem1, osem0, osem1,
             s_isem, s_osem):
        cid = lax.axis_index("c")
        sid = lax.axis_index("s")
        wid = sid * NC + cid
        base = wid * ROWS_PER_W

        class Pipe:
            def __init__(self, r_base, n, chunk, bufs, isems, osems):
                self.r_base, self.n, self.chunk = r_base, n, chunk
                self.bufs, self.isems, self.osems = bufs, isems, osems
                self.in_h = [None] * n
                self.out_h = [[] for _ in range(n)]
                self.step = 0

            def start_in(self, i):
                r0 = self.r_base + i * self.chunk
                self.in_h[i] = pltpu.async_copy(
                    table_hbm.at[pl.ds(r0, self.chunk)],
                    self.bufs[i % len(self.bufs)],
                    self.isems[i % len(self.isems)])

            def advance(self):
                i = self.step
                if i >= self.n:
                    return False
                cur = i % len(self.bufs)
                self.in_h[i].wait()
                r0 = self.r_base + i * self.chunk
                for b in range(B):
                    self.out_h[i].append(pltpu.async_copy(
                        self.bufs[cur],
                        out_hbm.at[b, pl.ds(r0, self.chunk)],
                        self.osems[i % len(self.osems)]))
                if i + 1 < self.n:
                    if i >= 1:
                        for h in self.out_h[i - 1]:
                            h.wait()
                    self.start_in(i + 1)
                self.step += 1
                return True

            def drain(self):
                for i in range(max(0, self.n - 2), self.n):
                    for h in self.out_h[i]:
                        h.wait()

        stream_pipe = Pipe(base, N_ST, CHUNK, (buf0, buf1),
                           (isem0, isem1), (osem0, osem1))
        spmem_pipe = Pipe(base + STREAM_ROWS, 1, SPMEM_ROWS,
                          (shb.at[sid],), (s_isem,), (s_osem,))

        stream_pipe.start_in(0)
        spmem_pipe.start_in(0)
        alive = True
        while alive:
            alive = False
            alive |= spmem_pipe.advance()
            alive |= stream_pipe.advance()
        spmem_pipe.drain()
        stream_pipe.drain()

    return body


_sc_copy = _make_sc_copy()


def kernel(x, pos_table):
    del x
    return _sc_copy(pos_table)


# speedup vs baseline: 1.0328x; 1.0328x over previous
import functools

import jax
import jax.numpy as jnp
from jax import lax
from jax.experimental import pallas as pl
from jax.experimental.pallas import tpu as pltpu
from jax.experimental.pallas import tpu_sc as plsc

NC = 2
NS = 16
NW = NC * NS

B = 4
S = 8192
D = 1024
ROWS_PER_W = S // NW
CHUNK = 32

STREAM_ROWS = 192
SPMEM_ROWS = ROWS_PER_W - STREAM_ROWS
N_ST = STREAM_ROWS // CHUNK
N_SP = SPMEM_ROWS // CHUNK


def _make_sc_copy():
    mesh = plsc.VectorSubcoreMesh(core_axis_name="c", subcore_axis_name="s")

    @functools.partial(
        pl.kernel,
        out_type=jax.ShapeDtypeStruct((B, S, D), jnp.float32),
        mesh=mesh,
        scratch_types=[
            pltpu.VMEM((CHUNK, D), jnp.float32),
            pltpu.VMEM((CHUNK, D), jnp.float32),
            pltpu.VMEM_SHARED((NS, 2, CHUNK, D), jnp.float32),
            pltpu.SemaphoreType.DMA,
            pltpu.SemaphoreType.DMA,
            pltpu.SemaphoreType.DMA,
            pltpu.SemaphoreType.DMA,
            pltpu.SemaphoreType.DMA,
            pltpu.SemaphoreType.DMA,
            pltpu.SemaphoreType.DMA,
            pltpu.SemaphoreType.DMA,
        ],
    )
    def body(table_hbm, out_hbm, buf0, buf1, shb,
             isem0, isem1, osem0, osem1,
             s_isem0, s_isem1, s_osem0, s_osem1):
        cid = lax.axis_index("c")
        sid = lax.axis_index("s")
        wid = sid * NC + cid
        base = wid * ROWS_PER_W

        class Pipe:
            def __init__(self, r_base, n, chunk, bufs, isems, osems):
                self.r_base, self.n, self.chunk = r_base, n, chunk
                self.bufs, self.isems, self.osems = bufs, isems, osems
                self.in_h = [None] * n
                self.out_h = [[] for _ in range(n)]
                self.step = 0

            def start_in(self, i):
                r0 = self.r_base + i * self.chunk
                self.in_h[i] = pltpu.async_copy(
                    table_hbm.at[pl.ds(r0, self.chunk)],
                    self.bufs[i % len(self.bufs)],
                    self.isems[i % len(self.isems)])

            def advance(self):
                i = self.step
                if i >= self.n:
                    return False
                cur = i % len(self.bufs)
                self.in_h[i].wait()
                r0 = self.r_base + i * self.chunk
                for b in range(B):
                    self.out_h[i].append(pltpu.async_copy(
                        self.bufs[cur],
                        out_hbm.at[b, pl.ds(r0, self.chunk)],
                        self.osems[i % len(self.osems)]))
                if i + 1 < self.n:
                    if i >= 1:
                        for h in self.out_h[i - 1]:
                            h.wait()
                    self.start_in(i + 1)
                self.step += 1
                return True

            def drain(self):
                for i in range(max(0, self.n - 2), self.n):
                    for h in self.out_h[i]:
                        h.wait()

        stream_pipe = Pipe(base, N_ST, CHUNK, (buf0, buf1),
                           (isem0, isem1), (osem0, osem1))
        spmem_pipe = Pipe(base + STREAM_ROWS, N_SP, CHUNK,
                          (shb.at[sid, 0], shb.at[sid, 1]),
                          (s_isem0, s_isem1), (s_osem0, s_osem1))

        stream_pipe.start_in(0)
        spmem_pipe.start_in(0)
        alive = True
        while alive:
            alive = False
            alive |= spmem_pipe.advance()
            alive |= stream_pipe.advance()
        spmem_pipe.drain()
        stream_pipe.drain()

    return body


_sc_copy = _make_sc_copy()


def kernel(x, pos_table):
    del x
    return _sc_copy(pos_table)
